# baseline (speedup 1.0000x reference)
import functools

import jax
import jax.numpy as jnp
from jax import lax
from jax.experimental import pallas as pl
from jax.experimental.pallas import tpu as pltpu
from jax.experimental.pallas import tpu_sc as plsc

NUM_Q = 4
K = 8192
D = 64
T = 8192

TBV = 128
CHW = 64
NCH = K // CHW

NC = 2
NS = 16
NW = NC * NS
TPW = T // NW
IDX_ROWS = T // 128
ROWS_PER_W = IDX_ROWS // NW


def _argmin_body(r_ref, cb_ref, rn_ref, cnb_ref, out_ref):
    r2t = (jnp.transpose(r_ref[...], (1, 0)) * 2.0).astype(jnp.bfloat16)
    rn_row = jnp.transpose(rn_ref[...], (1, 0))
    best = jnp.full((CHW, TBV), jnp.inf, jnp.float32)
    bestc = jnp.zeros((CHW, TBV), jnp.int32)
    for j in range(NCH):
        cbc = cb_ref[0, j * CHW:(j + 1) * CHW, :]
        m2 = lax.dot_general(
            cbc, r2t,
            dimension_numbers=(((1,), (0,)), ((), ())),
            preferred_element_type=jnp.float32,
        )
        s2 = (rn_row - m2) + cnb_ref[j * CHW:(j + 1) * CHW, :]
        upd = s2 < best
        best = jnp.where(upd, s2, best)
        bestc = jnp.where(upd, j, bestc)
    pos = lax.broadcasted_iota(jnp.int32, (CHW, TBV), 0)
    kidx = bestc * CHW + pos
    gmin = jnp.min(best, axis=0, keepdims=True)
    ind = jnp.min(jnp.where(best == gmin, kidx, jnp.int32(2**30)), axis=0)
    out_ref[0, 0, :] = ind


def _tc_argmin(r, cb16, rn, cnb, q):
    return pl.pallas_call(
        _argmin_body,
        grid=(T // TBV,),
        in_specs=[
            pl.BlockSpec((TBV, D), lambda i: (i, 0)),
            pl.BlockSpec((1, K, D), lambda i, _q=q: (_q, 0, 0)),
            pl.BlockSpec((TBV, 1), lambda i: (i, 0)),
            pl.BlockSpec((K, 128), lambda i, _q=q: (_q, 0)),
        ],
        out_specs=pl.BlockSpec((1, 1, TBV), lambda i: (i, 0, 0)),
        out_shape=jax.ShapeDtypeStruct((T // TBV, 1, TBV), jnp.int32),
        compiler_params=pltpu.CompilerParams(
            dimension_semantics=("arbitrary",),
        ),
    )(r, cb16, rn, cnb)


def _sc_update_body(cb_hbm, idx_hbm, r_hbm, out_hbm, idx_v, rows_v, r_v, o_v, sem):
    wid = lax.axis_index("s") * NC + lax.axis_index("c")
    pltpu.sync_copy(idx_hbm.at[pl.ds(wid * ROWS_PER_W, ROWS_PER_W)], idx_v)
    for j in range(ROWS_PER_W):
        pltpu.async_copy(
            cb_hbm.at[idx_v.at[j]], rows_v.at[pl.ds(j * 128, 128)], sem
        ).wait()
    pltpu.sync_copy(r_hbm.at[pl.ds(wid * TPW, TPW)], r_v)

    def body(i, _):
        q_row = rows_v.at[i]
        r_row = r_v.at[i]
        o_row = o_v.at[i]
        for j in range(D // 16):
            sl = pl.ds(j * 16, 16)
            qv = q_row[sl]
            rv = r_row[sl]
            q2 = rv + (qv - rv)
            o_row[sl] = rv - q2
        return 0

    lax.fori_loop(0, TPW, body, 0)
    pltpu.sync_copy(o_v, out_hbm.at[pl.ds(wid * TPW, TPW)])


@functools.cache
def _sc_update():
    return pl.kernel(
        _sc_update_body,
        out_type=jax.ShapeDtypeStruct((T, D), jnp.float32),
        mesh=plsc.VectorSubcoreMesh(core_axis_name="c", subcore_axis_name="s"),
        scratch_types=[
            pltpu.VMEM((ROWS_PER_W, 128), jnp.int32),
            pltpu.VMEM((TPW, 128), jnp.float32),
            pltpu.VMEM((TPW, D), jnp.float32),
            pltpu.VMEM((TPW, D), jnp.float32),
            pltpu.SemaphoreType.DMA,
        ],
    )


def kernel(x, codebooks):
    r = x.reshape(T, D)
    cb16 = codebooks.astype(jnp.bfloat16)
    cn = jnp.sum(codebooks**2, axis=-1)
    cnb = jnp.broadcast_to(cn.reshape(NUM_Q * K, 1), (NUM_Q * K, 128))
    cbp = jnp.pad(codebooks, ((0, 0), (0, 0), (0, 128 - D)))
    inds = []
    for q in range(NUM_Q):
        rn = jnp.sum(r**2, axis=-1, keepdims=True)
        ind = _tc_argmin(r, cb16, rn, cnb, q)
        r = _sc_update()(cbp[q], ind.reshape(IDX_ROWS, 128), r)
        inds.append(ind.reshape(8, 1024))
    quantized_out = x - r.reshape(x.shape)
    indices = jnp.stack(inds, axis=-1)
    return quantized_out, indices


# speedup vs baseline: 1.1210x; 1.1210x over previous
import functools

import jax
import jax.numpy as jnp
from jax import lax
from jax.experimental import pallas as pl
from jax.experimental.pallas import tpu as pltpu
from jax.experimental.pallas import tpu_sc as plsc

NUM_Q = 4
K = 8192
D = 64
T = 8192

TBV = 256
CHW = 32
NCH = K // CHW

NC = 2
NS = 16
NW = NC * NS
TPW = T // NW
IDX_ROWS = T // 128
ROWS_PER_W = IDX_ROWS // NW


def _argmin_body(r_ref, cb_ref, rn_ref, cnb_ref, out_ref):
    r2t = (jnp.transpose(r_ref[...], (1, 0)) * 2.0).astype(jnp.bfloat16)
    rn_row = jnp.transpose(rn_ref[...], (1, 0))
    best = jnp.full((CHW, TBV), jnp.inf, jnp.float32)
    bestc = jnp.zeros((CHW, TBV), jnp.int32)
    for j in range(NCH):
        cbc = cb_ref[0, j * CHW:(j + 1) * CHW, :]
        m2 = lax.dot_general(
            cbc, r2t,
            dimension_numbers=(((1,), (0,)), ((), ())),
            preferred_element_type=jnp.float32,
        )
        cnc = cnb_ref[j * CHW:(j + 1) * CHW, :]
        s2 = (rn_row - m2) + jnp.concatenate([cnc, cnc], axis=1)
        upd = s2 < best
        best = jnp.where(upd, s2, best)
        bestc = jnp.where(upd, j, bestc)
    pos = lax.broadcasted_iota(jnp.int32, (CHW, TBV), 0)
    kidx = bestc * CHW + pos
    gmin = jnp.min(best, axis=0, keepdims=True)
    ind = jnp.min(jnp.where(best == gmin, kidx, jnp.int32(2**30)), axis=0)
    out_ref[0, 0, :] = ind


def _tc_argmin(r, cb16, rn, cnb, q):
    return pl.pallas_call(
        _argmin_body,
        grid=(T // TBV,),
        in_specs=[
            pl.BlockSpec((TBV, D), lambda i: (i, 0)),
            pl.BlockSpec((1, K, D), lambda i, _q=q: (_q, 0, 0)),
            pl.BlockSpec((TBV, 1), lambda i: (i, 0)),
            pl.BlockSpec((K, 128), lambda i, _q=q: (_q, 0)),
        ],
        out_specs=pl.BlockSpec((1, 1, TBV), lambda i: (i, 0, 0)),
        out_shape=jax.ShapeDtypeStruct((T // TBV, 1, TBV), jnp.int32),
        compiler_params=pltpu.CompilerParams(
            dimension_semantics=("arbitrary",),
        ),
    )(r, cb16, rn, cnb)


def _sc_update_body(cb_hbm, idx_hbm, r_hbm, out_hbm, idx_v, rows_v, r_v, o_v, sem):
    wid = lax.axis_index("s") * NC + lax.axis_index("c")
    pltpu.sync_copy(idx_hbm.at[pl.ds(wid * ROWS_PER_W, ROWS_PER_W)], idx_v)
    for j in range(ROWS_PER_W):
        pltpu.async_copy(
            cb_hbm.at[idx_v.at[j]], rows_v.at[pl.ds(j * 128, 128)], sem
        ).wait()
    pltpu.sync_copy(r_hbm.at[pl.ds(wid * TPW, TPW)], r_v)

    def body(i, _):
        q_row = rows_v.at[i]
        r_row = r_v.at[i]
        o_row = o_v.at[i]
        for j in range(D // 16):
            sl = pl.ds(j * 16, 16)
            qv = q_row[sl]
            rv = r_row[sl]
            q2 = rv + (qv - rv)
            o_row[sl] = rv - q2
        return 0

    lax.fori_loop(0, TPW, body, 0)
    pltpu.sync_copy(o_v, out_hbm.at[pl.ds(wid * TPW, TPW)])


@functools.cache
def _sc_update():
    return pl.kernel(
        _sc_update_body,
        out_type=jax.ShapeDtypeStruct((T, D), jnp.float32),
        mesh=plsc.VectorSubcoreMesh(core_axis_name="c", subcore_axis_name="s"),
        scratch_types=[
            pltpu.VMEM((ROWS_PER_W, 128), jnp.int32),
            pltpu.VMEM((TPW, 128), jnp.float32),
            pltpu.VMEM((TPW, D), jnp.float32),
            pltpu.VMEM((TPW, D), jnp.float32),
            pltpu.SemaphoreType.DMA,
        ],
    )


def kernel(x, codebooks):
    r = x.reshape(T, D)
    cb16 = codebooks.astype(jnp.bfloat16)
    cn = jnp.sum(codebooks**2, axis=-1)
    cnb = jnp.broadcast_to(cn.reshape(NUM_Q * K, 1), (NUM_Q * K, 128))
    cbp = jnp.pad(codebooks, ((0, 0), (0, 0), (0, 128 - D)))
    inds = []
    for q in range(NUM_Q):
        rn = jnp.sum(r**2, axis=-1, keepdims=True)
        ind = _tc_argmin(r, cb16, rn, cnb, q)
        r = _sc_update()(cbp[q], ind.reshape(IDX_ROWS, 128), r)
        inds.append(ind.reshape(8, 1024))
    quantized_out = x - r.reshape(x.shape)
    indices = jnp.stack(inds, axis=-1)
    return quantized_out, indices
